# baseline (speedup 1.0000x reference)
import functools

import jax
import jax.numpy as jnp
from jax import lax
from jax.experimental import pallas as pl
from jax.experimental.pallas import tpu as pltpu
from jax.experimental.pallas import tpu_sc as plsc

SIGMA = 1.0
EPSILON = 1.0
CUTOFF = 2.5

NUM_CORES = 2
NUM_SUBCORES = 16
NUM_TILES = NUM_CORES * NUM_SUBCORES
LANES = 16
C_EDGES = 2048
K_SUB = C_EDGES // 128
GRPS = C_EDGES // LANES


def _round_up(x, m):
    return (x + m - 1) // m * m


@functools.partial(jax.jit, static_argnames=("n_pad", "n_chunks"))
def _lj_call(px, py, pz, zf, mi2d, mj2d, n_pad, n_chunks):
    rows_stage = n_pad // NUM_SUBCORES

    def body(*refs):
        (px_hbm, py_hbm, pz_hbm, zf_hbm, mi_hbm, mj_hbm, fpart, epart,
         px_sp, py_sp, pz_sp, fx_sp, fy_sp, fz_sp) = refs[:14]
        scr = refs[14:]
        ibs = tuple(scr[2 * b] for b in range(3))
        jbs = tuple(scr[2 * b + 1] for b in range(3))
        gbs = tuple(tuple(scr[6 + 6 * b + k] for k in range(6))
                    for b in range(3))
        fbs = tuple(tuple(scr[24 + 6 * b + k] for k in range(6))
                    for b in range(3))
        ev, bounce = scr[42:44]
        gsems = scr[44:47]
        ssems = scr[47:50]

        cid = lax.axis_index("c")
        sid = lax.axis_index("s")
        wid = cid * NUM_SUBCORES + sid
        r0 = sid * rows_stage
        sl = pl.ds(r0, rows_stage)
        for src_hbm, dst_sp in ((px_hbm, px_sp), (py_hbm, py_sp),
                                (pz_hbm, pz_sp), (zf_hbm, fx_sp),
                                (zf_hbm, fy_sp), (zf_hbm, fz_sp)):
            pltpu.sync_copy(src_hbm.at[sl], bounce)
            pltpu.sync_copy(bounce, dst_sp.at[sl])
        ev[...] = jnp.zeros((LANES,), jnp.float32)
        plsc.subcore_barrier()

        def fetch_fire(c, b):
            ebase = (wid * n_chunks + c) * C_EDGES
            pltpu.sync_copy(mi_hbm.at[pl.ds(ebase, C_EDGES)], ibs[b])
            pltpu.sync_copy(mj_hbm.at[pl.ds(ebase, C_EDGES)], jbs[b])
            xbi, ybi, zbi, xbj, ybj, zbj = gbs[b]
            pltpu.async_copy(px_sp.at[ibs[b]], xbi, gsems[b])
            pltpu.async_copy(py_sp.at[ibs[b]], ybi, gsems[b])
            pltpu.async_copy(pz_sp.at[ibs[b]], zbi, gsems[b])
            pltpu.async_copy(px_sp.at[jbs[b]], xbj, gsems[b])
            pltpu.async_copy(py_sp.at[jbs[b]], ybj, gsems[b])
            pltpu.async_copy(pz_sp.at[jbs[b]], zbj, gsems[b])

        def drain_gathers(b):
            xbi, ybi, zbi, xbj, ybj, zbj = gbs[b]
            for dst in (xbi, ybi, zbi, xbj, ybj, zbj):
                pltpu.make_async_copy(px_sp.at[ibs[b]], dst, gsems[b]).wait()

        def fire_scatters(b):
            fxi, fyi, fzi, fxj, fyj, fzj = fbs[b]
            pltpu.async_copy(fxi, fx_sp.at[ibs[b]], ssems[b], add=True)
            pltpu.async_copy(fyi, fy_sp.at[ibs[b]], ssems[b], add=True)
            pltpu.async_copy(fzi, fz_sp.at[ibs[b]], ssems[b], add=True)
            pltpu.async_copy(fxj, fx_sp.at[jbs[b]], ssems[b], add=True)
            pltpu.async_copy(fyj, fy_sp.at[jbs[b]], ssems[b], add=True)
            pltpu.async_copy(fzj, fz_sp.at[jbs[b]], ssems[b], add=True)

        def drain_scatters(b):
            fxi, fyi, fzi, fxj, fyj, fzj = fbs[b]
            pltpu.make_async_copy(fxi, fx_sp.at[ibs[b]], ssems[b]).wait()
            pltpu.make_async_copy(fyi, fy_sp.at[ibs[b]], ssems[b]).wait()
            pltpu.make_async_copy(fzi, fz_sp.at[ibs[b]], ssems[b]).wait()
            pltpu.make_async_copy(fxj, fx_sp.at[jbs[b]], ssems[b]).wait()
            pltpu.make_async_copy(fyj, fy_sp.at[jbs[b]], ssems[b]).wait()
            pltpu.make_async_copy(fzj, fz_sp.at[jbs[b]], ssems[b]).wait()

        def compute(b):
            xbi, ybi, zbi, xbj, ybj, zbj = gbs[b]
            fxi, fyi, fzi, fxj, fyj, fzj = fbs[b]

            @plsc.parallel_loop(0, C_EDGES, step=LANES, unroll=2,
                                carry=jnp.zeros((LANES,), jnp.float32))
            def acc(g, e_acc):
                v = pl.ds(g, LANES)
                dx = xbj[v] - xbi[v]
                dy = ybj[v] - ybi[v]
                dz = zbj[v] - zbi[v]
                d2 = dx * dx + dy * dy + dz * dz
                valid = (d2 > 0.0) & (d2 < CUTOFF * CUTOFF)
                d2s = jnp.where(valid, d2, 1.0)
                inv = 1.0 / d2s
                inv3 = inv * inv * inv
                inv6 = inv3 * inv3
                e = jnp.where(valid, 4.0 * EPSILON * (inv6 - inv3), 0.0)
                fs = jnp.where(valid,
                               (24.0 * EPSILON * inv) * (2.0 * inv6 - inv3),
                               0.0)
                fx = fs * dx
                fy = fs * dy
                fz = fs * dz
                fxj[v] = fx
                fyj[v] = fy
                fzj[v] = fz
                fxi[v] = -fx
                fyi[v] = -fy
                fzi[v] = -fz
                return e_acc + e

            ev[...] = ev[...] + acc

        fetch_fire(0, 0)

        @pl.loop(0, n_chunks, step=3)
        def _trip(c):
            for b in (0, 1, 2):
                cc = c + b
                nxt = (b + 1) % 3

                @pl.when(cc + 1 < n_chunks)
                def _():
                    @pl.when(cc >= 2)
                    def _():
                        drain_scatters(nxt)
                    fetch_fire(cc + 1, nxt)

                drain_gathers(b)
                compute(b)
                fire_scatters(b)

        drain_scatters(0)
        drain_scatters(1)
        drain_scatters(2)
        plsc.subcore_barrier()
        base = cid * 3 * n_pad + r0
        for k, src_sp in enumerate((fx_sp, fy_sp, fz_sp)):
            pltpu.sync_copy(src_sp.at[sl], bounce)
            pltpu.sync_copy(bounce, fpart.at[pl.ds(base + k * n_pad,
                                                   rows_stage)])
        pltpu.sync_copy(ev, epart.at[pl.ds(wid * LANES, LANES)])

    mesh = plsc.VectorSubcoreMesh(core_axis_name="c", subcore_axis_name="s")
    fpart, epart = pl.kernel(
        body,
        out_type=[
            jax.ShapeDtypeStruct((NUM_CORES * 3 * n_pad,), jnp.float32),
            jax.ShapeDtypeStruct((NUM_TILES * LANES,), jnp.float32),
        ],
        mesh=mesh,
        scratch_types=(
            [pltpu.VMEM_SHARED((n_pad,), jnp.float32)] * 6
            + [pltpu.VMEM((C_EDGES,), jnp.int32)] * 6
            + [pltpu.VMEM((C_EDGES,), jnp.float32)] * 36
            + [pltpu.VMEM((LANES,), jnp.float32),
               pltpu.VMEM((n_pad // NUM_SUBCORES,), jnp.float32)]
            + [pltpu.SemaphoreType.DMA] * 6
        ),
    )(px, py, pz, zf, mi2d, mj2d)
    return fpart, epart


def kernel(positions, mapping):
    n = positions.shape[0]
    n_edges = mapping.shape[1]
    n_pad = _round_up(n, 128)
    trip = 3 * NUM_TILES * C_EDGES
    e_pad = trip * ((n_edges + trip - 1) // trip)
    n_chunks = e_pad // (NUM_TILES * C_EDGES)

    pos_pad = jnp.zeros((3, n_pad), jnp.float32).at[:, :n].set(positions.T)
    zf = jnp.zeros((n_pad,), jnp.float32)
    mi = jnp.zeros((e_pad,), jnp.int32).at[:n_edges].set(mapping[0])
    mj = jnp.zeros((e_pad,), jnp.int32).at[:n_edges].set(mapping[1])

    fpart, epart = _lj_call(pos_pad[0], pos_pad[1], pos_pad[2], zf,
                            mi, mj, n_pad, n_chunks)
    energy = 0.5 * jnp.sum(epart)
    fp = fpart.reshape(NUM_CORES, 3, n_pad)
    forces = (fp[0] + fp[1]).T[:n]
    return (energy, forces)


# speedup vs baseline: 2.2707x; 2.2707x over previous
import functools

import jax
import jax.numpy as jnp
from jax import lax
from jax.experimental import pallas as pl
from jax.experimental.pallas import tpu as pltpu
from jax.experimental.pallas import tpu_sc as plsc

SIGMA = 1.0
EPSILON = 1.0
CUTOFF = 2.5

NUM_CORES = 2
NUM_SUBCORES = 16
NUM_TILES = NUM_CORES * NUM_SUBCORES
LANES = 16
C_EDGES = 2048
K_SUB = C_EDGES // 128
GRPS = C_EDGES // LANES


def _round_up(x, m):
    return (x + m - 1) // m * m


@functools.partial(jax.jit, static_argnames=("n_pad", "n_chunks"))
def _lj_call(px, py, pz, zf, mi2d, mj2d, n_pad, n_chunks):
    rows_stage = n_pad // NUM_SUBCORES

    def body(*refs):
        (px_hbm, py_hbm, pz_hbm, zf_hbm, mi_hbm, mj_hbm, fpart, epart,
         px_sp, py_sp, pz_sp, fx_sp, fy_sp, fz_sp) = refs[:14]
        scr = refs[14:]
        ibs = tuple(scr[2 * b] for b in range(3))
        jbs = tuple(scr[2 * b + 1] for b in range(3))
        gbs = tuple(tuple(scr[6 + 6 * b + k] for k in range(6))
                    for b in range(3))
        fbs = tuple(tuple(scr[24 + 6 * b + k] for k in range(6))
                    for b in range(3))
        ev, bounce = scr[42:44]
        gsems = scr[44:47]
        ssems = scr[47:50]

        cid = lax.axis_index("c")
        sid = lax.axis_index("s")
        wid = cid * NUM_SUBCORES + sid
        r0 = sid * rows_stage
        sl = pl.ds(r0, rows_stage)
        for src_hbm, dst_sp in ((px_hbm, px_sp), (py_hbm, py_sp),
                                (pz_hbm, pz_sp), (zf_hbm, fx_sp),
                                (zf_hbm, fy_sp), (zf_hbm, fz_sp)):
            pltpu.sync_copy(src_hbm.at[sl], bounce)
            pltpu.sync_copy(bounce, dst_sp.at[sl])
        ev[...] = jnp.zeros((LANES,), jnp.float32)
        plsc.subcore_barrier()

        def fetch_fire(c, b):
            ebase = (wid * n_chunks + c) * C_EDGES
            pltpu.sync_copy(mi_hbm.at[pl.ds(ebase, C_EDGES)], ibs[b])
            pltpu.sync_copy(mj_hbm.at[pl.ds(ebase, C_EDGES)], jbs[b])
            xbi, ybi, zbi, xbj, ybj, zbj = gbs[b]
            pltpu.async_copy(px_sp.at[ibs[b]], xbi, gsems[b])
            pltpu.async_copy(py_sp.at[ibs[b]], ybi, gsems[b])
            pltpu.async_copy(pz_sp.at[ibs[b]], zbi, gsems[b])
            pltpu.async_copy(px_sp.at[jbs[b]], xbj, gsems[b])
            pltpu.async_copy(py_sp.at[jbs[b]], ybj, gsems[b])
            pltpu.async_copy(pz_sp.at[jbs[b]], zbj, gsems[b])

        def drain_gathers(b):
            xbi, ybi, zbi, xbj, ybj, zbj = gbs[b]
            for dst in (xbi, ybi, zbi, xbj, ybj, zbj):
                pltpu.make_async_copy(px_sp.at[ibs[b]], dst, gsems[b]).wait()

        def fire_scatters(b):
            fxi, fyi, fzi, fxj, fyj, fzj = fbs[b]
            pltpu.async_copy(fxi, fx_sp.at[ibs[b]], ssems[b], add=True)
            pltpu.async_copy(fyi, fy_sp.at[ibs[b]], ssems[b], add=True)
            pltpu.async_copy(fzi, fz_sp.at[ibs[b]], ssems[b], add=True)
            pltpu.async_copy(fxj, fx_sp.at[jbs[b]], ssems[b], add=True)
            pltpu.async_copy(fyj, fy_sp.at[jbs[b]], ssems[b], add=True)
            pltpu.async_copy(fzj, fz_sp.at[jbs[b]], ssems[b], add=True)

        def drain_scatters(b):
            fxi, fyi, fzi, fxj, fyj, fzj = fbs[b]
            pltpu.make_async_copy(fxi, fx_sp.at[ibs[b]], ssems[b]).wait()
            pltpu.make_async_copy(fyi, fy_sp.at[ibs[b]], ssems[b]).wait()
            pltpu.make_async_copy(fzi, fz_sp.at[ibs[b]], ssems[b]).wait()
            pltpu.make_async_copy(fxj, fx_sp.at[jbs[b]], ssems[b]).wait()
            pltpu.make_async_copy(fyj, fy_sp.at[jbs[b]], ssems[b]).wait()
            pltpu.make_async_copy(fzj, fz_sp.at[jbs[b]], ssems[b]).wait()

        def compute(b):
            xbi, ybi, zbi, xbj, ybj, zbj = gbs[b]
            fxi, fyi, fzi, fxj, fyj, fzj = fbs[b]

            @plsc.parallel_loop(0, C_EDGES, step=LANES, unroll=2,
                                carry=jnp.zeros((LANES,), jnp.float32))
            def acc(g, e_acc):
                v = pl.ds(g, LANES)
                dx = xbj[v] - xbi[v]
                dy = ybj[v] - ybi[v]
                dz = zbj[v] - zbi[v]
                d2 = dx * dx + dy * dy + dz * dz
                valid = (d2 > 0.0) & (d2 < CUTOFF * CUTOFF)
                d2s = jnp.where(valid, d2, 1.0)
                inv = 1.0 / d2s
                inv3 = inv * inv * inv
                inv6 = inv3 * inv3
                e = jnp.where(valid, 4.0 * EPSILON * (inv6 - inv3), 0.0)
                fs = jnp.where(valid,
                               (24.0 * EPSILON * inv) * (2.0 * inv6 - inv3),
                               0.0)
                fx = fs * dx
                fy = fs * dy
                fz = fs * dz
                fxj[v] = fx
                fyj[v] = fy
                fzj[v] = fz
                fxi[v] = -fx
                fyi[v] = -fy
                fzi[v] = -fz
                return e_acc + e

            ev[...] = ev[...] + acc

        fetch_fire(0, 0)

        @pl.loop(0, n_chunks, step=3)
        def _trip(c):
            for b in (0, 1, 2):
                cc = c + b
                nxt = (b + 1) % 3

                @pl.when(cc + 1 < n_chunks)
                def _():
                    @pl.when(cc >= 2)
                    def _():
                        drain_scatters(nxt)
                    fetch_fire(cc + 1, nxt)

                drain_gathers(b)
                compute(b)
                fire_scatters(b)

        drain_scatters(0)
        drain_scatters(1)
        drain_scatters(2)
        plsc.subcore_barrier()
        base = cid * 3 * n_pad + r0
        for k, src_sp in enumerate((fx_sp, fy_sp, fz_sp)):
            pltpu.sync_copy(src_sp.at[sl], bounce)
            pltpu.sync_copy(bounce, fpart.at[pl.ds(base + k * n_pad,
                                                   rows_stage)])
        pltpu.sync_copy(ev, epart.at[pl.ds(wid * LANES, LANES)])

    mesh = plsc.VectorSubcoreMesh(core_axis_name="c", subcore_axis_name="s")
    fpart, epart = pl.kernel(
        body,
        out_type=[
            jax.ShapeDtypeStruct((NUM_CORES * 3 * n_pad,), jnp.float32),
            jax.ShapeDtypeStruct((NUM_TILES * LANES,), jnp.float32),
        ],
        mesh=mesh,
        scratch_types=(
            [pltpu.VMEM_SHARED((n_pad,), jnp.float32)] * 6
            + [pltpu.VMEM((C_EDGES,), jnp.int32)] * 6
            + [pltpu.VMEM((C_EDGES,), jnp.float32)] * 36
            + [pltpu.VMEM((LANES,), jnp.float32),
               pltpu.VMEM((n_pad // NUM_SUBCORES,), jnp.float32)]
            + [pltpu.SemaphoreType.DMA] * 6
        ),
    )(px, py, pz, zf, mi2d, mj2d)
    return fpart, epart


def kernel(positions, mapping):
    n = positions.shape[0]
    n_edges = mapping.shape[1]
    n_pad = _round_up(n, 128)
    trip = 3 * NUM_TILES * C_EDGES
    e_pad = trip * ((n_edges + trip - 1) // trip)
    n_chunks = e_pad // (NUM_TILES * C_EDGES)

    pos_pad = jnp.zeros((3, n_pad), jnp.float32).at[:, :n].set(positions.T)
    zf = jnp.zeros((n_pad,), jnp.float32)
    spread = jnp.arange(e_pad, dtype=jnp.int32) % n
    mi = spread.at[:n_edges].set(mapping[0])
    mj = spread.at[:n_edges].set(mapping[1])

    fpart, epart = _lj_call(pos_pad[0], pos_pad[1], pos_pad[2], zf,
                            mi, mj, n_pad, n_chunks)
    energy = 0.5 * jnp.sum(epart)
    fp = fpart.reshape(NUM_CORES, 3, n_pad)
    forces = (fp[0] + fp[1]).T[:n]
    return (energy, forces)


# speedup vs baseline: 2.5160x; 1.1080x over previous
import functools

import jax
import jax.numpy as jnp
from jax import lax
from jax.experimental import pallas as pl
from jax.experimental.pallas import tpu as pltpu
from jax.experimental.pallas import tpu_sc as plsc

SIGMA = 1.0
EPSILON = 1.0
CUTOFF = 2.5

NUM_CORES = 2
NUM_SUBCORES = 16
NUM_TILES = NUM_CORES * NUM_SUBCORES
LANES = 16
C_EDGES = 2048
K_SUB = C_EDGES // 128
GRPS = C_EDGES // LANES


def _round_up(x, m):
    return (x + m - 1) // m * m


@functools.partial(jax.jit,
                   static_argnames=("n_pad", "n_chunks", "n_edges"))
def _lj_call(px, py, pz, zf, mi, mj, tail_i, tail_j,
             n_pad, n_chunks, n_edges):
    rows_stage = n_pad // NUM_SUBCORES
    full = n_edges // C_EDGES
    rem = n_edges % C_EDGES
    last = full if rem else full - 1

    def body(*refs):
        (px_hbm, py_hbm, pz_hbm, zf_hbm, mi_hbm, mj_hbm,
         ti_hbm, tj_hbm, fpart, epart,
         px_sp, py_sp, pz_sp, fx_sp, fy_sp, fz_sp) = refs[:16]
        scr = refs[16:]
        ibs = tuple(scr[2 * b] for b in range(3))
        jbs = tuple(scr[2 * b + 1] for b in range(3))
        gbs = tuple(tuple(scr[6 + 6 * b + k] for k in range(6))
                    for b in range(3))
        fbs = tuple(tuple(scr[24 + 6 * b + k] for k in range(6))
                    for b in range(3))
        ev, bounce = scr[42:44]
        gsems = scr[44:47]
        ssems = scr[47:50]

        cid = lax.axis_index("c")
        sid = lax.axis_index("s")
        wid = cid * NUM_SUBCORES + sid
        r0 = sid * rows_stage
        sl = pl.ds(r0, rows_stage)
        for src_hbm, dst_sp in ((px_hbm, px_sp), (py_hbm, py_sp),
                                (pz_hbm, pz_sp), (zf_hbm, fx_sp),
                                (zf_hbm, fy_sp), (zf_hbm, fz_sp)):
            pltpu.sync_copy(src_hbm.at[sl], bounce)
            pltpu.sync_copy(bounce, dst_sp.at[sl])
        ev[...] = jnp.zeros((LANES,), jnp.float32)
        plsc.subcore_barrier()

        def chunk_id(c):
            return c * NUM_TILES + wid

        def fetch_fire(c, b):
            t = chunk_id(c)

            @pl.when(t < full)
            def _():
                ebase = t * C_EDGES
                pltpu.sync_copy(mi_hbm.at[pl.ds(ebase, C_EDGES)], ibs[b])
                pltpu.sync_copy(mj_hbm.at[pl.ds(ebase, C_EDGES)], jbs[b])

            if rem:
                @pl.when(t == full)
                def _():
                    pltpu.sync_copy(ti_hbm, ibs[b])
                    pltpu.sync_copy(tj_hbm, jbs[b])

            xbi, ybi, zbi, xbj, ybj, zbj = gbs[b]
            pltpu.async_copy(px_sp.at[ibs[b]], xbi, gsems[b])
            pltpu.async_copy(py_sp.at[ibs[b]], ybi, gsems[b])
            pltpu.async_copy(pz_sp.at[ibs[b]], zbi, gsems[b])
            pltpu.async_copy(px_sp.at[jbs[b]], xbj, gsems[b])
            pltpu.async_copy(py_sp.at[jbs[b]], ybj, gsems[b])
            pltpu.async_copy(pz_sp.at[jbs[b]], zbj, gsems[b])

        def drain_gathers(b):
            xbi, ybi, zbi, xbj, ybj, zbj = gbs[b]
            for dst in (xbi, ybi, zbi, xbj, ybj, zbj):
                pltpu.make_async_copy(px_sp.at[ibs[b]], dst, gsems[b]).wait()

        def fire_scatters(b):
            fxi, fyi, fzi, fxj, fyj, fzj = fbs[b]
            pltpu.async_copy(fxi, fx_sp.at[ibs[b]], ssems[b], add=True)
            pltpu.async_copy(fyi, fy_sp.at[ibs[b]], ssems[b], add=True)
            pltpu.async_copy(fzi, fz_sp.at[ibs[b]], ssems[b], add=True)
            pltpu.async_copy(fxj, fx_sp.at[jbs[b]], ssems[b], add=True)
            pltpu.async_copy(fyj, fy_sp.at[jbs[b]], ssems[b], add=True)
            pltpu.async_copy(fzj, fz_sp.at[jbs[b]], ssems[b], add=True)

        def drain_scatters(b):
            fxi, fyi, fzi, fxj, fyj, fzj = fbs[b]
            pltpu.make_async_copy(fxi, fx_sp.at[ibs[b]], ssems[b]).wait()
            pltpu.make_async_copy(fyi, fy_sp.at[ibs[b]], ssems[b]).wait()
            pltpu.make_async_copy(fzi, fz_sp.at[ibs[b]], ssems[b]).wait()
            pltpu.make_async_copy(fxj, fx_sp.at[jbs[b]], ssems[b]).wait()
            pltpu.make_async_copy(fyj, fy_sp.at[jbs[b]], ssems[b]).wait()
            pltpu.make_async_copy(fzj, fz_sp.at[jbs[b]], ssems[b]).wait()

        def compute(b):
            xbi, ybi, zbi, xbj, ybj, zbj = gbs[b]
            fxi, fyi, fzi, fxj, fyj, fzj = fbs[b]

            @plsc.parallel_loop(0, C_EDGES, step=LANES, unroll=2,
                                carry=jnp.zeros((LANES,), jnp.float32))
            def acc(g, e_acc):
                v = pl.ds(g, LANES)
                dx = xbj[v] - xbi[v]
                dy = ybj[v] - ybi[v]
                dz = zbj[v] - zbi[v]
                d2 = dx * dx + dy * dy + dz * dz
                valid = (d2 > 0.0) & (d2 < CUTOFF * CUTOFF)
                d2s = jnp.where(valid, d2, 1.0)
                inv = 1.0 / d2s
                inv3 = inv * inv * inv
                inv6 = inv3 * inv3
                e = jnp.where(valid, 4.0 * EPSILON * (inv6 - inv3), 0.0)
                fs = jnp.where(valid,
                               (24.0 * EPSILON * inv) * (2.0 * inv6 - inv3),
                               0.0)
                fx = fs * dx
                fy = fs * dy
                fz = fs * dz
                fxj[v] = fx
                fyj[v] = fy
                fzj[v] = fz
                fxi[v] = -fx
                fyi[v] = -fy
                fzi[v] = -fz
                return e_acc + e

            ev[...] = ev[...] + acc

        fetch_fire(0, 0)

        @pl.loop(0, n_chunks, step=3)
        def _trip(c):
            for b in (0, 1, 2):
                cc = c + b
                nxt = (b + 1) % 3

                @pl.when(chunk_id(cc + 1) <= last)
                def _():
                    @pl.when(cc >= 2)
                    def _():
                        drain_scatters(nxt)
                    fetch_fire(cc + 1, nxt)

                @pl.when(chunk_id(cc) <= last)
                def _():
                    drain_gathers(b)
                    compute(b)
                    fire_scatters(b)

        drain_scatters(0)
        drain_scatters(1)
        drain_scatters(2)
        plsc.subcore_barrier()
        base = cid * 3 * n_pad + r0
        for k, src_sp in enumerate((fx_sp, fy_sp, fz_sp)):
            pltpu.sync_copy(src_sp.at[sl], bounce)
            pltpu.sync_copy(bounce, fpart.at[pl.ds(base + k * n_pad,
                                                   rows_stage)])
        pltpu.sync_copy(ev, epart.at[pl.ds(wid * LANES, LANES)])

    mesh = plsc.VectorSubcoreMesh(core_axis_name="c", subcore_axis_name="s")
    fpart, epart = pl.kernel(
        body,
        out_type=[
            jax.ShapeDtypeStruct((NUM_CORES * 3 * n_pad,), jnp.float32),
            jax.ShapeDtypeStruct((NUM_TILES * LANES,), jnp.float32),
        ],
        mesh=mesh,
        scratch_types=(
            [pltpu.VMEM_SHARED((n_pad,), jnp.float32)] * 6
            + [pltpu.VMEM((C_EDGES,), jnp.int32)] * 6
            + [pltpu.VMEM((C_EDGES,), jnp.float32)] * 36
            + [pltpu.VMEM((LANES,), jnp.float32),
               pltpu.VMEM((n_pad // NUM_SUBCORES,), jnp.float32)]
            + [pltpu.SemaphoreType.DMA] * 6
        ),
    )(px, py, pz, zf, mi, mj, tail_i, tail_j)
    return fpart, epart


def kernel(positions, mapping):
    n = positions.shape[0]
    n_edges = mapping.shape[1]
    n_pad = _round_up(n, 128)
    total_chunks = (n_edges + C_EDGES - 1) // C_EDGES
    n_chunks = _round_up((total_chunks + NUM_TILES - 1) // NUM_TILES, 3)

    pos_pad = jnp.zeros((3, n_pad), jnp.float32).at[:, :n].set(positions.T)
    zf = jnp.zeros((n_pad,), jnp.float32)
    full = n_edges // C_EDGES
    rem = n_edges % C_EDGES
    spread = jnp.arange(C_EDGES, dtype=jnp.int32) % n
    if rem:
        keep = jnp.arange(C_EDGES) < rem
        tail_i = jnp.where(
            keep, jnp.pad(mapping[0, full * C_EDGES:], (0, C_EDGES - rem)),
            spread)
        tail_j = jnp.where(
            keep, jnp.pad(mapping[1, full * C_EDGES:], (0, C_EDGES - rem)),
            spread)
    else:
        tail_i = tail_j = spread

    fpart, epart = _lj_call(pos_pad[0], pos_pad[1], pos_pad[2], zf,
                            mapping[0], mapping[1], tail_i, tail_j,
                            n_pad, n_chunks, n_edges)
    energy = 0.5 * jnp.sum(epart)
    fp = fpart.reshape(NUM_CORES, 3, n_pad)
    forces = (fp[0] + fp[1]).T[:n]
    return (energy, forces)


# speedup vs baseline: 2.5694x; 1.0212x over previous
import functools

import jax
import jax.numpy as jnp
from jax import lax
from jax.experimental import pallas as pl
from jax.experimental.pallas import tpu as pltpu
from jax.experimental.pallas import tpu_sc as plsc

SIGMA = 1.0
EPSILON = 1.0
CUTOFF = 2.5

NUM_CORES = 2
NUM_SUBCORES = 16
NUM_TILES = NUM_CORES * NUM_SUBCORES
LANES = 16
C_EDGES = 2048
K_SUB = C_EDGES // 128
GRPS = C_EDGES // LANES


def _round_up(x, m):
    return (x + m - 1) // m * m


@functools.partial(jax.jit,
                   static_argnames=("n_pad", "n_chunks", "n_edges"))
def _lj_call(px, py, pz, zf, mflat, tail_i, tail_j,
             n_pad, n_chunks, n_edges):
    rows_stage = n_pad // NUM_SUBCORES
    full = n_edges // C_EDGES
    rem = n_edges % C_EDGES
    last = full if rem else full - 1

    def body(*refs):
        (px_hbm, py_hbm, pz_hbm, zf_hbm, mflat_hbm,
         ti_hbm, tj_hbm, fpart, epart,
         px_sp, py_sp, pz_sp, fx_sp, fy_sp, fz_sp) = refs[:15]
        scr = refs[15:]
        ibs = tuple(scr[2 * b] for b in range(3))
        jbs = tuple(scr[2 * b + 1] for b in range(3))
        gbs = tuple(tuple(scr[6 + 6 * b + k] for k in range(6))
                    for b in range(3))
        fbs = tuple(tuple(scr[24 + 6 * b + k] for k in range(6))
                    for b in range(3))
        ev, bounce = scr[42:44]
        gsems = scr[44:47]
        ssems = scr[47:50]

        cid = lax.axis_index("c")
        sid = lax.axis_index("s")
        wid = cid * NUM_SUBCORES + sid
        r0 = sid * rows_stage
        sl = pl.ds(r0, rows_stage)
        for src_hbm, dst_sp in ((px_hbm, px_sp), (py_hbm, py_sp),
                                (pz_hbm, pz_sp), (zf_hbm, fx_sp),
                                (zf_hbm, fy_sp), (zf_hbm, fz_sp)):
            pltpu.sync_copy(src_hbm.at[sl], bounce)
            pltpu.sync_copy(bounce, dst_sp.at[sl])
        ev[...] = jnp.zeros((LANES,), jnp.float32)
        plsc.subcore_barrier()

        def chunk_id(c):
            return c * NUM_TILES + wid

        def fetch_fire(c, b):
            t = chunk_id(c)

            @pl.when(t < full)
            def _():
                ebase = t * C_EDGES
                pltpu.sync_copy(mflat_hbm.at[pl.ds(ebase, C_EDGES)],
                                ibs[b])
                pltpu.sync_copy(mflat_hbm.at[pl.ds(n_edges + ebase,
                                                   C_EDGES)], jbs[b])

            if rem:
                @pl.when(t == full)
                def _():
                    pltpu.sync_copy(ti_hbm, ibs[b])
                    pltpu.sync_copy(tj_hbm, jbs[b])

            xbi, ybi, zbi, xbj, ybj, zbj = gbs[b]
            pltpu.async_copy(px_sp.at[ibs[b]], xbi, gsems[b])
            pltpu.async_copy(py_sp.at[ibs[b]], ybi, gsems[b])
            pltpu.async_copy(pz_sp.at[ibs[b]], zbi, gsems[b])
            pltpu.async_copy(px_sp.at[jbs[b]], xbj, gsems[b])
            pltpu.async_copy(py_sp.at[jbs[b]], ybj, gsems[b])
            pltpu.async_copy(pz_sp.at[jbs[b]], zbj, gsems[b])

        def drain_gathers(b):
            xbi, ybi, zbi, xbj, ybj, zbj = gbs[b]
            for dst in (xbi, ybi, zbi, xbj, ybj, zbj):
                pltpu.make_async_copy(px_sp.at[ibs[b]], dst, gsems[b]).wait()

        def fire_scatters(b):
            fxi, fyi, fzi, fxj, fyj, fzj = fbs[b]
            pltpu.async_copy(fxi, fx_sp.at[ibs[b]], ssems[b], add=True)
            pltpu.async_copy(fyi, fy_sp.at[ibs[b]], ssems[b], add=True)
            pltpu.async_copy(fzi, fz_sp.at[ibs[b]], ssems[b], add=True)
            pltpu.async_copy(fxj, fx_sp.at[jbs[b]], ssems[b], add=True)
            pltpu.async_copy(fyj, fy_sp.at[jbs[b]], ssems[b], add=True)
            pltpu.async_copy(fzj, fz_sp.at[jbs[b]], ssems[b], add=True)

        def drain_scatters(b):
            fxi, fyi, fzi, fxj, fyj, fzj = fbs[b]
            pltpu.make_async_copy(fxi, fx_sp.at[ibs[b]], ssems[b]).wait()
            pltpu.make_async_copy(fyi, fy_sp.at[ibs[b]], ssems[b]).wait()
            pltpu.make_async_copy(fzi, fz_sp.at[ibs[b]], ssems[b]).wait()
            pltpu.make_async_copy(fxj, fx_sp.at[jbs[b]], ssems[b]).wait()
            pltpu.make_async_copy(fyj, fy_sp.at[jbs[b]], ssems[b]).wait()
            pltpu.make_async_copy(fzj, fz_sp.at[jbs[b]], ssems[b]).wait()

        def compute(b):
            xbi, ybi, zbi, xbj, ybj, zbj = gbs[b]
            fxi, fyi, fzi, fxj, fyj, fzj = fbs[b]

            @plsc.parallel_loop(0, C_EDGES, step=LANES, unroll=4,
                                carry=jnp.zeros((LANES,), jnp.float32))
            def acc(g, e_acc):
                v = pl.ds(g, LANES)
                dx = xbj[v] - xbi[v]
                dy = ybj[v] - ybi[v]
                dz = zbj[v] - zbi[v]
                d2 = dx * dx + dy * dy + dz * dz
                valid = (d2 > 0.0) & (d2 < CUTOFF * CUTOFF)
                d2s = jnp.where(valid, d2, 1.0)
                inv = 1.0 / d2s
                inv3 = inv * inv * inv
                inv6 = inv3 * inv3
                e = jnp.where(valid, 4.0 * EPSILON * (inv6 - inv3), 0.0)
                fs = jnp.where(valid,
                               (24.0 * EPSILON * inv) * (2.0 * inv6 - inv3),
                               0.0)
                fx = fs * dx
                fy = fs * dy
                fz = fs * dz
                fxj[v] = fx
                fyj[v] = fy
                fzj[v] = fz
                fxi[v] = -fx
                fyi[v] = -fy
                fzi[v] = -fz
                return e_acc + e

            ev[...] = ev[...] + acc

        fetch_fire(0, 0)

        @pl.loop(0, n_chunks, step=3)
        def _trip(c):
            for b in (0, 1, 2):
                cc = c + b
                nxt = (b + 1) % 3

                @pl.when(chunk_id(cc + 1) <= last)
                def _():
                    @pl.when(cc >= 2)
                    def _():
                        drain_scatters(nxt)
                    fetch_fire(cc + 1, nxt)

                @pl.when(chunk_id(cc) <= last)
                def _():
                    drain_gathers(b)
                    compute(b)
                    fire_scatters(b)

        drain_scatters(0)
        drain_scatters(1)
        drain_scatters(2)
        plsc.subcore_barrier()
        base = cid * 3 * n_pad + r0
        for k, src_sp in enumerate((fx_sp, fy_sp, fz_sp)):
            pltpu.sync_copy(src_sp.at[sl], bounce)
            pltpu.sync_copy(bounce, fpart.at[pl.ds(base + k * n_pad,
                                                   rows_stage)])
        pltpu.sync_copy(ev, epart.at[pl.ds(wid * LANES, LANES)])

    mesh = plsc.VectorSubcoreMesh(core_axis_name="c", subcore_axis_name="s")
    fpart, epart = pl.kernel(
        body,
        out_type=[
            jax.ShapeDtypeStruct((NUM_CORES * 3 * n_pad,), jnp.float32),
            jax.ShapeDtypeStruct((NUM_TILES * LANES,), jnp.float32),
        ],
        mesh=mesh,
        scratch_types=(
            [pltpu.VMEM_SHARED((n_pad,), jnp.float32)] * 6
            + [pltpu.VMEM((C_EDGES,), jnp.int32)] * 6
            + [pltpu.VMEM((C_EDGES,), jnp.float32)] * 36
            + [pltpu.VMEM((LANES,), jnp.float32),
               pltpu.VMEM((n_pad // NUM_SUBCORES,), jnp.float32)]
            + [pltpu.SemaphoreType.DMA] * 6
        ),
    )(px, py, pz, zf, mflat, tail_i, tail_j)
    return fpart, epart


def kernel(positions, mapping):
    n = positions.shape[0]
    n_edges = mapping.shape[1]
    n_pad = _round_up(n, 128)
    total_chunks = (n_edges + C_EDGES - 1) // C_EDGES
    n_chunks = _round_up((total_chunks + NUM_TILES - 1) // NUM_TILES, 3)

    pos_pad = jnp.zeros((3, n_pad), jnp.float32).at[:, :n].set(positions.T)
    zf = jnp.zeros((n_pad,), jnp.float32)
    full = n_edges // C_EDGES
    rem = n_edges % C_EDGES
    spread = jnp.arange(C_EDGES, dtype=jnp.int32) % n
    if rem:
        keep = jnp.arange(C_EDGES) < rem
        tail_i = jnp.where(
            keep, jnp.pad(mapping[0, full * C_EDGES:], (0, C_EDGES - rem)),
            spread)
        tail_j = jnp.where(
            keep, jnp.pad(mapping[1, full * C_EDGES:], (0, C_EDGES - rem)),
            spread)
    else:
        tail_i = tail_j = spread

    fpart, epart = _lj_call(pos_pad[0], pos_pad[1], pos_pad[2], zf,
                            mapping.reshape(-1), tail_i, tail_j,
                            n_pad, n_chunks, n_edges)
    energy = 0.5 * jnp.sum(epart)
    fp = fpart.reshape(NUM_CORES, 3, n_pad)
    forces = (fp[0] + fp[1]).T[:n]
    return (energy, forces)
